# Initial kernel scaffold; baseline (speedup 1.0000x reference)
import functools

import jax
import jax.numpy as jnp
from jax.experimental import pallas as pl
from jax.experimental.pallas import tpu as pltpu

_CONV_DIMS = [(64, 128), (128, 256), (256, 512)]

_BLKA = 256
_BLKB = 32


def _leaky(x):
    return jnp.where(x >= 0, x, 0.01 * x)


def _mlp_body(trees_ref, *refs):
    w_refs = refs[:8]
    b_refs = refs[8:16]
    out_ref = refs[16]
    x = trees_ref[...]
    for w, b in zip(w_refs, b_refs):
        x = _leaky(jnp.dot(x, w[...], preferred_element_type=jnp.float32)
                   + b[...])
    out_ref[...] = x


def _conv_body(x_ref, idx_ref, *refs, nblk):
    w_refs = refs[:3]
    b_refs = refs[3:6]
    out_ref = refs[6]
    x = x_ref[...]
    idx = idx_ref[...]
    for i, (cin, cout) in enumerate(_CONV_DIMS):
        x3 = x.reshape(nblk, 64, cin)
        g = jnp.take_along_axis(x3, idx[:, :, None], axis=1)
        g2 = g.reshape(nblk * 63, 3 * cin)
        h = jnp.dot(g2, w_refs[i][...], preferred_element_type=jnp.float32)
        h = h + b_refs[i][...]
        h3 = h.reshape(nblk, 63, cout)
        z = jnp.concatenate(
            [jnp.zeros((nblk, 1, cout), jnp.float32), h3], axis=1)
        xf = z.reshape(nblk * 64, cout)
        s1 = jnp.sum(xf, axis=1, keepdims=True)
        s2 = jnp.sum(xf * xf, axis=1, keepdims=True)
        t1 = jnp.sum(s1.reshape(nblk, 64), axis=1, keepdims=True)
        t2 = jnp.sum(s2.reshape(nblk, 64), axis=1, keepdims=True)
        n = 64.0 * cout
        mean = t1 / n
        var = (t2 - t1 * t1 / n) / (n - 1.0)
        denom = jnp.sqrt(var) + 1e-5
        meanr = jnp.broadcast_to(mean, (nblk, 64)).reshape(nblk * 64, 1)
        denr = jnp.broadcast_to(denom, (nblk, 64)).reshape(nblk * 64, 1)
        x = _leaky((xf - meanr) / denr)
    out_ref[...] = jnp.transpose(x.reshape(nblk, 64, 512), (0, 2, 1))


@jax.jit
def kernel(trees, indexes, lw0, lb0, lw1, lb1, lw2, lb2, lw3, lb3, lw4, lb4,
           lw5, lb5, lw6, lb6, lw7, lb7, cw0, cb0, cw1, cb1, cw2, cb2):
    B = trees.shape[0]
    lw7p = lw7.reshape(2048, 64, 64).swapaxes(1, 2).reshape(2048, 4096)
    lb7p = lb7.reshape(64, 64).swapaxes(0, 1).reshape(4096)
    lws = [lw0, lw1, lw2, lw3, lw4, lw5, lw6, lw7p]
    lbs = [b.reshape(1, -1)
           for b in (lb0, lb1, lb2, lb3, lb4, lb5, lb6, lb7p)]

    grid_a = B // _BLKA
    y = pl.pallas_call(
        _mlp_body,
        grid=(grid_a,),
        in_specs=[pl.BlockSpec((_BLKA, 16), lambda i: (i, 0))]
        + [pl.BlockSpec(w.shape, lambda i: (0, 0)) for w in lws]
        + [pl.BlockSpec(b.shape, lambda i: (0, 0)) for b in lbs],
        out_specs=pl.BlockSpec((_BLKA, 4096), lambda i: (i, 0)),
        out_shape=jax.ShapeDtypeStruct((B, 4096), jnp.float32),
        compiler_params=pltpu.CompilerParams(
            dimension_semantics=("arbitrary",)),
    )(trees, *lws, *lbs)

    x2 = y.reshape(B * 64, 64)
    idx2 = indexes[:, :, 0]
    cws = [cw.transpose(2, 1, 0).reshape(3 * ci, co)
           for (ci, co), cw in zip(_CONV_DIMS, (cw0, cw1, cw2))]
    cbs = [cb.reshape(1, -1) for cb in (cb0, cb1, cb2)]

    grid_b = B // _BLKB
    out = pl.pallas_call(
        functools.partial(_conv_body, nblk=_BLKB),
        grid=(grid_b,),
        in_specs=[pl.BlockSpec((_BLKB * 64, 64), lambda i: (i, 0)),
                  pl.BlockSpec((_BLKB, 189), lambda i: (i, 0))]
        + [pl.BlockSpec(w.shape, lambda i: (0, 0)) for w in cws]
        + [pl.BlockSpec(b.shape, lambda i: (0, 0)) for b in cbs],
        out_specs=pl.BlockSpec((_BLKB, 512, 64), lambda i: (i, 0, 0)),
        out_shape=jax.ShapeDtypeStruct((B, 512, 64), jnp.float32),
        compiler_params=pltpu.CompilerParams(
            dimension_semantics=("arbitrary",)),
    )(x2, idx2, *cws, *cbs)

    return (out, indexes)


# speedup vs baseline: 760.3760x; 760.3760x over previous
import functools

import jax
import jax.numpy as jnp
from jax.experimental import pallas as pl
from jax.experimental.pallas import tpu as pltpu

_CONV_DIMS = [(64, 128), (128, 256), (256, 512)]

_BLKA = 256
_BLKB = 32


def _leaky(x):
    return jnp.where(x >= 0, x, 0.01 * x)


def _mlp_body(trees_ref, *refs):
    w_refs = refs[:8]
    b_refs = refs[8:16]
    out_ref = refs[16]
    x = trees_ref[...]
    for w, b in zip(w_refs, b_refs):
        x = _leaky(jnp.dot(x, w[...], preferred_element_type=jnp.float32)
                   + b[...])
    out_ref[...] = x


def _conv_body(x_ref, i0_ref, i1_ref, i2_ref, *refs, nblk):
    w_refs = refs[:9]
    b_refs = refs[9:12]
    out_ref = refs[12]
    x3 = x_ref[...].reshape(nblk, 64, 64)
    idx_k = [i0_ref[...], i1_ref[...], i2_ref[...]]
    for i, (cin, cout) in enumerate(_CONV_DIMS):
        h = jnp.broadcast_to(b_refs[i][...], (nblk * 63, cout))
        for k in range(3):
            g = jnp.take_along_axis(
                x3,
                jnp.broadcast_to(idx_k[k][:, None, :], (nblk, cin, 63)),
                axis=2)
            gt = jnp.swapaxes(g, 1, 2)
            h = h + jnp.dot(gt.reshape(nblk * 63, cin), w_refs[3 * i + k][...],
                            preferred_element_type=jnp.float32)
        h3 = h.reshape(nblk, 63, cout)
        z = jnp.concatenate(
            [jnp.zeros((nblk, 1, cout), jnp.float32), h3], axis=1)
        t1 = jnp.sum(jnp.sum(z, axis=2), axis=1, keepdims=True)
        t2 = jnp.sum(jnp.sum(z * z, axis=2), axis=1, keepdims=True)
        n = 64.0 * cout
        mean = t1 / n
        var = (t2 - t1 * t1 / n) / (n - 1.0)
        rden = 1.0 / (jnp.sqrt(var) + 1e-5)
        xn = _leaky((z - mean[:, :, None]) * rden[:, :, None])
        x3 = jnp.swapaxes(xn, 1, 2)
    out_ref[...] = x3


@jax.jit
def kernel(trees, indexes, lw0, lb0, lw1, lb1, lw2, lb2, lw3, lb3, lw4, lb4,
           lw5, lb5, lw6, lb6, lw7, lb7, cw0, cb0, cw1, cb1, cw2, cb2):
    B = trees.shape[0]
    lws = [lw0, lw1, lw2, lw3, lw4, lw5, lw6, lw7]
    lbs = [b.reshape(1, -1)
           for b in (lb0, lb1, lb2, lb3, lb4, lb5, lb6, lb7)]

    grid_a = B // _BLKA
    y = pl.pallas_call(
        _mlp_body,
        grid=(grid_a,),
        in_specs=[pl.BlockSpec((_BLKA, 16), lambda i: (i, 0))]
        + [pl.BlockSpec(w.shape, lambda i: (0, 0)) for w in lws]
        + [pl.BlockSpec(b.shape, lambda i: (0, 0)) for b in lbs],
        out_specs=pl.BlockSpec((_BLKA, 4096), lambda i: (i, 0)),
        out_shape=jax.ShapeDtypeStruct((B, 4096), jnp.float32),
        compiler_params=pltpu.CompilerParams(
            dimension_semantics=("arbitrary",)),
    )(trees, *lws, *lbs)

    idx3 = indexes.reshape(B, 63, 3)
    idx_ks = [idx3[:, :, k] for k in range(3)]
    cws = [cw[:, :, k].swapaxes(0, 1)
           for cw in (cw0, cw1, cw2) for k in range(3)]
    cbs = [cb.reshape(1, -1) for cb in (cb0, cb1, cb2)]

    grid_b = B // _BLKB
    out = pl.pallas_call(
        functools.partial(_conv_body, nblk=_BLKB),
        grid=(grid_b,),
        in_specs=[pl.BlockSpec((_BLKB, 4096), lambda i: (i, 0))]
        + [pl.BlockSpec((_BLKB, 63), lambda i: (i, 0)) for _ in range(3)]
        + [pl.BlockSpec(w.shape, lambda i: (0, 0)) for w in cws]
        + [pl.BlockSpec(b.shape, lambda i: (0, 0)) for b in cbs],
        out_specs=pl.BlockSpec((_BLKB, 512, 64), lambda i: (i, 0, 0)),
        out_shape=jax.ShapeDtypeStruct((B, 512, 64), jnp.float32),
        compiler_params=pltpu.CompilerParams(
            dimension_semantics=("arbitrary",)),
    )(y, *idx_ks, *cws, *cbs)

    return (out, indexes)


# speedup vs baseline: 1215.2711x; 1.5983x over previous
import functools

import jax
import jax.numpy as jnp
from jax.experimental import pallas as pl
from jax.experimental.pallas import tpu as pltpu

_CONV_DIMS = [(64, 128), (128, 256), (256, 512)]

_BLKA = 512
_BLKB = 64


def _leaky(x):
    return jnp.where(x >= 0, x, 0.01 * x)


def _mlp_body(trees_ref, *refs):
    w_refs = refs[:8]
    b_refs = refs[8:16]
    out_ref = refs[16]
    x = trees_ref[...]
    for w, b in zip(w_refs, b_refs):
        x = _leaky(jnp.dot(x.astype(jnp.bfloat16), w[...],
                           preferred_element_type=jnp.float32) + b[...])
    out_ref[...] = x


def _conv_body(x_ref, i0_ref, i1_ref, i2_ref, *refs, nblk):
    w_refs = refs[:3]
    b_refs = refs[3:6]
    out_ref = refs[6]
    x3 = x_ref[...].reshape(nblk, 64, 64)
    idx_k = [i0_ref[...], i1_ref[...], i2_ref[...]]
    node0 = jax.lax.broadcasted_iota(jnp.int32, (nblk, 64, 1), 1) == 0
    for i, (cin, cout) in enumerate(_CONV_DIMS):
        parts = []
        for k in range(3):
            g = jnp.take_along_axis(
                x3,
                jnp.broadcast_to(idx_k[k][:, None, :], (nblk, cin, 64)),
                axis=2)
            gt = jnp.swapaxes(g.astype(jnp.bfloat16), 1, 2)
            parts.append(gt.reshape(nblk * 64, cin))
        gcat = jnp.concatenate(parts, axis=1)
        h = jnp.dot(gcat, w_refs[i][...],
                    preferred_element_type=jnp.float32) + b_refs[i][...]
        z = jnp.where(node0, 0.0, h.reshape(nblk, 64, cout))
        t1 = jnp.sum(jnp.sum(z, axis=2), axis=1, keepdims=True)
        t2 = jnp.sum(jnp.sum(z * z, axis=2), axis=1, keepdims=True)
        n = 64.0 * cout
        mean = t1 / n
        var = (t2 - t1 * t1 / n) / (n - 1.0)
        rden = 1.0 / (jnp.sqrt(var) + 1e-5)
        xn = _leaky((z - mean[:, :, None]) * rden[:, :, None])
        if i < 2:
            x3 = jnp.swapaxes(xn, 1, 2)
        else:
            out_ref[...] = jnp.swapaxes(xn, 1, 2)


@jax.jit
def kernel(trees, indexes, lw0, lb0, lw1, lb1, lw2, lb2, lw3, lb3, lw4, lb4,
           lw5, lb5, lw6, lb6, lw7, lb7, cw0, cb0, cw1, cb1, cw2, cb2):
    B = trees.shape[0]
    lws = [w.astype(jnp.bfloat16)
           for w in (lw0, lw1, lw2, lw3, lw4, lw5, lw6, lw7)]
    lbs = [b.reshape(1, -1)
           for b in (lb0, lb1, lb2, lb3, lb4, lb5, lb6, lb7)]

    grid_a = B // _BLKA
    y = pl.pallas_call(
        _mlp_body,
        grid=(grid_a,),
        in_specs=[pl.BlockSpec((_BLKA, 16), lambda i: (i, 0))]
        + [pl.BlockSpec(w.shape, lambda i: (0, 0)) for w in lws]
        + [pl.BlockSpec(b.shape, lambda i: (0, 0)) for b in lbs],
        out_specs=pl.BlockSpec((_BLKA, 4096), lambda i: (i, 0)),
        out_shape=jax.ShapeDtypeStruct((B, 4096), jnp.float32),
        compiler_params=pltpu.CompilerParams(
            dimension_semantics=("arbitrary",)),
    )(trees, *lws, *lbs)

    idx3 = indexes.reshape(B, 63, 3)
    zcol = jnp.zeros((B, 1), jnp.int32)
    idx_ks = [jnp.concatenate([zcol, idx3[:, :, k]], axis=1)
              for k in range(3)]
    cws = [cw.transpose(2, 1, 0).reshape(3 * ci, co).astype(jnp.bfloat16)
           for (ci, co), cw in zip(_CONV_DIMS, (cw0, cw1, cw2))]
    cbs = [cb.reshape(1, -1) for cb in (cb0, cb1, cb2)]

    grid_b = B // _BLKB
    out = pl.pallas_call(
        functools.partial(_conv_body, nblk=_BLKB),
        grid=(grid_b,),
        in_specs=[pl.BlockSpec((_BLKB, 4096), lambda i: (i, 0))]
        + [pl.BlockSpec((_BLKB, 64), lambda i: (i, 0)) for _ in range(3)]
        + [pl.BlockSpec(w.shape, lambda i: (0, 0)) for w in cws]
        + [pl.BlockSpec(b.shape, lambda i: (0, 0)) for b in cbs],
        out_specs=pl.BlockSpec((_BLKB, 512, 64), lambda i: (i, 0, 0)),
        out_shape=jax.ShapeDtypeStruct((B, 512, 64), jnp.float32),
        compiler_params=pltpu.CompilerParams(
            dimension_semantics=("arbitrary",)),
    )(y, *idx_ks, *cws, *cbs)

    return (out, indexes)


# speedup vs baseline: 1330.8175x; 1.0951x over previous
import functools

import jax
import jax.numpy as jnp
from jax.experimental import pallas as pl
from jax.experimental.pallas import tpu as pltpu

_CONV_DIMS = [(64, 128), (128, 256), (256, 512)]

_BLKA = 512
_BLKB = 64


def _leaky(x):
    return jnp.where(x >= 0, x, 0.01 * x)


def _mlp_body(trees_ref, *refs):
    w_refs = refs[:8]
    b_refs = refs[8:16]
    out_ref = refs[16]
    x = trees_ref[...]
    for w, b in zip(w_refs, b_refs):
        x = _leaky(jnp.dot(x.astype(jnp.bfloat16), w[...],
                           preferred_element_type=jnp.float32) + b[...])
    out_ref[...] = x


def _conv_body(x_ref, i0_ref, i1_ref, i2_ref, *refs, nblk):
    w_refs = refs[:3]
    b_refs = refs[3:6]
    out_ref = refs[6]
    x3 = x_ref[...].reshape(nblk, 64, 64)
    idx_k = [i0_ref[...], i1_ref[...], i2_ref[...]]
    node0 = jax.lax.broadcasted_iota(jnp.int32, (nblk, 64, 1), 1) == 0
    for i, (cin, cout) in enumerate(_CONV_DIMS):
        parts = []
        for k in range(3):
            g = jnp.take_along_axis(
                x3,
                jnp.broadcast_to(idx_k[k][:, None, :], (nblk, cin, 64)),
                axis=2)
            gt = jnp.swapaxes(g.astype(jnp.bfloat16), 1, 2)
            parts.append(gt.reshape(nblk * 64, cin))
        gcat = jnp.concatenate(parts, axis=1)
        h = jnp.dot(gcat, w_refs[i][...],
                    preferred_element_type=jnp.float32) + b_refs[i][...]
        z = jnp.where(node0, 0.0, h.reshape(nblk, 64, cout))
        t1 = jnp.sum(z, axis=(1, 2), keepdims=True)[:, :, 0]
        t2 = jnp.sum(z * z, axis=(1, 2), keepdims=True)[:, :, 0]
        n = 64.0 * cout
        mean = t1 / n
        var = (t2 - t1 * t1 / n) / (n - 1.0)
        rden = 1.0 / (jnp.sqrt(var) + 1e-5)
        xn = _leaky((z - mean[:, :, None]) * rden[:, :, None])
        if i < 2:
            x3 = jnp.swapaxes(xn, 1, 2)
        else:
            out_ref[...] = jnp.swapaxes(
                xn.astype(jnp.bfloat16), 1, 2).astype(jnp.float32)


@jax.jit
def kernel(trees, indexes, lw0, lb0, lw1, lb1, lw2, lb2, lw3, lb3, lw4, lb4,
           lw5, lb5, lw6, lb6, lw7, lb7, cw0, cb0, cw1, cb1, cw2, cb2):
    B = trees.shape[0]
    lws = [w.astype(jnp.bfloat16)
           for w in (lw0, lw1, lw2, lw3, lw4, lw5, lw6, lw7)]
    lbs = [b.reshape(1, -1)
           for b in (lb0, lb1, lb2, lb3, lb4, lb5, lb6, lb7)]

    grid_a = B // _BLKA
    y = pl.pallas_call(
        _mlp_body,
        grid=(grid_a,),
        in_specs=[pl.BlockSpec((_BLKA, 16), lambda i: (i, 0))]
        + [pl.BlockSpec(w.shape, lambda i: (0, 0)) for w in lws]
        + [pl.BlockSpec(b.shape, lambda i: (0, 0)) for b in lbs],
        out_specs=pl.BlockSpec((_BLKA, 4096), lambda i: (i, 0)),
        out_shape=jax.ShapeDtypeStruct((B, 4096), jnp.float32),
        compiler_params=pltpu.CompilerParams(
            dimension_semantics=("arbitrary",)),
    )(trees, *lws, *lbs)

    idx3 = indexes.reshape(B, 63, 3)
    zcol = jnp.zeros((B, 1), jnp.int32)
    idx_ks = [jnp.concatenate([zcol, idx3[:, :, k]], axis=1)
              for k in range(3)]
    cws = [cw.transpose(2, 1, 0).reshape(3 * ci, co).astype(jnp.bfloat16)
           for (ci, co), cw in zip(_CONV_DIMS, (cw0, cw1, cw2))]
    cbs = [cb.reshape(1, -1) for cb in (cb0, cb1, cb2)]

    grid_b = B // _BLKB
    out = pl.pallas_call(
        functools.partial(_conv_body, nblk=_BLKB),
        grid=(grid_b,),
        in_specs=[pl.BlockSpec((_BLKB, 4096), lambda i: (i, 0))]
        + [pl.BlockSpec((_BLKB, 64), lambda i: (i, 0)) for _ in range(3)]
        + [pl.BlockSpec(w.shape, lambda i: (0, 0)) for w in cws]
        + [pl.BlockSpec(b.shape, lambda i: (0, 0)) for b in cbs],
        out_specs=pl.BlockSpec((_BLKB, 512, 64), lambda i: (i, 0, 0)),
        out_shape=jax.ShapeDtypeStruct((B, 512, 64), jnp.float32),
        compiler_params=pltpu.CompilerParams(
            dimension_semantics=("arbitrary",)),
    )(y, *idx_ks, *cws, *cbs)

    return (out, indexes)
